# baseline (speedup 1.0000x reference)
import jax
import jax.numpy as jnp
from jax.experimental import pallas as pl
from jax.experimental.pallas import tpu as pltpu

_F32 = jnp.float32
_BT = 256


def _lenet_kernel(x_ref, w1c_ref, w2c_ref, w1m_ref, w2m_ref, bias_ref,
                  o_ref, qp):
    w1c = w1c_ref[...]
    bias = bias_ref[...]
    b1p = bias[0:96]
    b2p = bias[96:224]
    b1m = bias[224:736]
    b2m = bias[736:746]
    z28 = jnp.zeros((28, _BT), _F32)
    slab = [z28, z28] + [x_ref[j] for j in range(28)] + [z28, z28]
    qp[0:2] = jnp.zeros((2, 96, _BT), _F32)
    qp[16:18] = jnp.zeros((2, 96, _BT), _F32)
    for w2 in range(14):
        ra = jnp.concatenate([slab[2 * w2 + kw] for kw in range(5)], axis=0)
        rb = jnp.concatenate([slab[2 * w2 + 1 + kw] for kw in range(5)],
                             axis=0)
        ya = jnp.dot(w1c, ra, preferred_element_type=_F32)
        yb = jnp.dot(w1c, rb, preferred_element_type=_F32)
        m = jnp.maximum(jnp.maximum(ya[:96], ya[96:]),
                        jnp.maximum(yb[:96], yb[96:]))
        qp[2 + w2] = jnp.maximum(m + b1p, 0.0)

    w2c = w2c_ref[...]
    cols = []
    for w4 in range(7):
        ra = jnp.concatenate([qp[2 * w4 + kw] for kw in range(5)], axis=0)
        rb = jnp.concatenate([qp[2 * w4 + 1 + kw] for kw in range(5)], axis=0)
        ya = jnp.dot(w2c, ra, preferred_element_type=_F32)
        yb = jnp.dot(w2c, rb, preferred_element_type=_F32)
        m = jnp.maximum(jnp.maximum(ya[:128], ya[128:]),
                        jnp.maximum(yb[:128], yb[128:]))
        cols.append(jnp.maximum(m + b2p, 0.0))
    f = jnp.concatenate(cols, axis=0)

    h = jnp.dot(w1m_ref[...], f, preferred_element_type=_F32)
    h = jnp.maximum(h + b1m, 0.0)
    y = jnp.dot(w2m_ref[...], h, preferred_element_type=_F32) + b2m
    m = jnp.max(y, axis=0, keepdims=True)
    z = y - m
    o_ref[...] = jnp.transpose(
        z - jnp.log(jnp.sum(jnp.exp(z), axis=0, keepdims=True)))


def _band_part(wk, p, n_out, real_out, n_in, real_in):
    oc_n, ic_n = wk.shape[0], wk.shape[1]
    h2 = jnp.arange(n_out)
    hp = jnp.arange(n_in)
    band = jnp.zeros((oc_n, n_out, 5, ic_n, n_in), _F32)
    for kh in range(5):
        e = ((hp[None, :] - (2 * h2[:, None] + p) == kh - 2)
             & (h2[:, None] < real_out)
             & (hp[None, :] < real_in)).astype(_F32)
        band = band + (jnp.transpose(wk[:, :, kh, :], (0, 2, 1))
                       [:, None, :, :, None] * e[None, :, None, None, :])
    return band.reshape(oc_n * n_out, 5 * ic_n * n_in)


def _band_eo(wk, n_out, real_out, n_in, real_in):
    return jnp.concatenate(
        [_band_part(wk, 0, n_out, real_out, n_in, real_in),
         _band_part(wk, 1, n_out, real_out, n_in, real_in)], axis=0)


def kernel(x_nchw, conv1_wmat, conv1_b, conv2_wmat, conv2_b,
           fc1_wt, fc1_b, fc2_wt, fc2_b):
    B = x_nchw.shape[0]
    grid = B // _BT

    xT = jnp.transpose(x_nchw.reshape(B, 28, 28), (2, 1, 0))

    w1 = jnp.transpose(conv1_wmat[:25, :6]).reshape(6, 1, 5, 5)
    w2 = jnp.transpose(conv2_wmat[:150, :16].reshape(5, 5, 6, 16),
                       (3, 2, 0, 1))
    w1c = _band_eo(w1, 16, 14, 28, 28)
    w2c = _band_eo(w2, 8, 7, 16, 14)
    hm1 = (jnp.arange(16) < 14).astype(_F32)
    hm2 = (jnp.arange(8) < 7).astype(_F32)
    bias = jnp.concatenate([
        (conv1_b[:6, None] * hm1).reshape(96),
        (conv2_b[:16, None] * hm2).reshape(128),
        fc1_b, fc2_b, jnp.zeros((22,), _F32)])
    bias = jnp.broadcast_to(bias[:, None], (768, _BT))

    w1m = fc1_wt.reshape(7, 7, 16, 512).transpose(1, 2, 0, 3)
    w1m = jnp.pad(w1m, ((0, 0), (0, 0), (0, 1), (0, 0)))
    w1m = w1m.reshape(896, 512).T
    w2m = fc2_wt.T

    return pl.pallas_call(
        _lenet_kernel,
        out_shape=jax.ShapeDtypeStruct((B, 10), _F32),
        grid=(grid,),
        in_specs=[
            pl.BlockSpec((28, 28, _BT), lambda i: (0, 0, i)),
            pl.BlockSpec((192, 140), lambda i: (0, 0)),
            pl.BlockSpec((256, 480), lambda i: (0, 0)),
            pl.BlockSpec((512, 896), lambda i: (0, 0)),
            pl.BlockSpec((10, 512), lambda i: (0, 0)),
            pl.BlockSpec((768, _BT), lambda i: (0, 0)),
        ],
        out_specs=pl.BlockSpec((_BT, 10), lambda i: (i, 0)),
        scratch_shapes=[
            pltpu.VMEM((18, 96, _BT), _F32),
        ],
        compiler_params=pltpu.CompilerParams(
            dimension_semantics=("parallel",),
            vmem_limit_bytes=64 * 1024 * 1024,
        ),
    )(xT, w1c, w2c, w1m, w2m, bias)


# speedup vs baseline: 1.0525x; 1.0525x over previous
import jax
import jax.numpy as jnp
from jax.experimental import pallas as pl
from jax.experimental.pallas import tpu as pltpu

_F32 = jnp.float32
_BT = 256


def _lenet_kernel(x_ref, w1c_ref, w2c_ref, w1m_ref, w2m_ref, bias_ref,
                  o_ref, qp):
    w1c = w1c_ref[...]
    bias = bias_ref[...]
    b1p = bias[0:96]
    b2p = bias[96:224]
    b1m = bias[224:736]
    b2m = bias[736:746]
    z28 = jnp.zeros((28, _BT), _F32)
    slab = [z28, z28] + [x_ref[j] for j in range(28)] + [z28, z28]
    qp[0:2] = jnp.zeros((2, 96, _BT), _F32)
    qp[16:18] = jnp.zeros((2, 96, _BT), _F32)
    for w2 in range(14):
        ra = jnp.concatenate([slab[2 * w2 + kw] for kw in range(5)], axis=0)
        rb = jnp.concatenate([slab[2 * w2 + 1 + kw] for kw in range(5)],
                             axis=0)
        ya = jnp.dot(w1c, ra, preferred_element_type=_F32)
        yb = jnp.dot(w1c, rb, preferred_element_type=_F32)
        m = jnp.maximum(jnp.maximum(ya[:96], ya[96:]),
                        jnp.maximum(yb[:96], yb[96:]))
        qp[2 + w2] = jnp.maximum(m + b1p, 0.0)

    w2c = w2c_ref[...]
    cols = []
    for w4 in range(7):
        ra = jnp.concatenate([qp[2 * w4 + kw] for kw in range(5)], axis=0)
        rb = jnp.concatenate([qp[2 * w4 + 1 + kw] for kw in range(5)], axis=0)
        ya = jnp.dot(w2c, ra, preferred_element_type=_F32)
        yb = jnp.dot(w2c, rb, preferred_element_type=_F32)
        m = jnp.maximum(jnp.maximum(ya[:128], ya[128:]),
                        jnp.maximum(yb[:128], yb[128:]))
        cols.append(jnp.maximum(m + b2p, 0.0))
    f = jnp.concatenate(cols, axis=0)

    h = jnp.dot(w1m_ref[...], f, preferred_element_type=_F32)
    h = jnp.maximum(h + b1m, 0.0)
    y = jnp.dot(w2m_ref[...], h, preferred_element_type=_F32) + b2m
    m = jnp.max(y, axis=0, keepdims=True)
    z = y - m
    o_ref[...] = z - jnp.log(jnp.sum(jnp.exp(z), axis=0, keepdims=True))


def _band_part(wk, p, n_out, real_out, n_in, real_in):
    oc_n, ic_n = wk.shape[0], wk.shape[1]
    h2 = jnp.arange(n_out)
    hp = jnp.arange(n_in)
    band = jnp.zeros((oc_n, n_out, 5, ic_n, n_in), _F32)
    for kh in range(5):
        e = ((hp[None, :] - (2 * h2[:, None] + p) == kh - 2)
             & (h2[:, None] < real_out)
             & (hp[None, :] < real_in)).astype(_F32)
        band = band + (jnp.transpose(wk[:, :, kh, :], (0, 2, 1))
                       [:, None, :, :, None] * e[None, :, None, None, :])
    return band.reshape(oc_n * n_out, 5 * ic_n * n_in)


def _band_eo(wk, n_out, real_out, n_in, real_in):
    return jnp.concatenate(
        [_band_part(wk, 0, n_out, real_out, n_in, real_in),
         _band_part(wk, 1, n_out, real_out, n_in, real_in)], axis=0)


def kernel(x_nchw, conv1_wmat, conv1_b, conv2_wmat, conv2_b,
           fc1_wt, fc1_b, fc2_wt, fc2_b):
    B = x_nchw.shape[0]
    grid = B // _BT

    xT = jnp.transpose(x_nchw.reshape(B, 28, 28), (2, 1, 0))

    w1 = jnp.transpose(conv1_wmat[:25, :6]).reshape(6, 1, 5, 5)
    w2 = jnp.transpose(conv2_wmat[:150, :16].reshape(5, 5, 6, 16),
                       (3, 2, 0, 1))
    w1c = _band_eo(w1, 16, 14, 28, 28)
    w2c = _band_eo(w2, 8, 7, 16, 14)
    hm1 = (jnp.arange(16) < 14).astype(_F32)
    hm2 = (jnp.arange(8) < 7).astype(_F32)
    bias = jnp.concatenate([
        (conv1_b[:6, None] * hm1).reshape(96),
        (conv2_b[:16, None] * hm2).reshape(128),
        fc1_b, fc2_b, jnp.zeros((22,), _F32)])
    bias = jnp.broadcast_to(bias[:, None], (768, _BT))

    w1m = fc1_wt.reshape(7, 7, 16, 512).transpose(1, 2, 0, 3)
    w1m = jnp.pad(w1m, ((0, 0), (0, 0), (0, 1), (0, 0)))
    w1m = w1m.reshape(896, 512).T
    w2m = fc2_wt.T

    out = pl.pallas_call(
        _lenet_kernel,
        out_shape=jax.ShapeDtypeStruct((10, B), _F32),
        grid=(grid,),
        in_specs=[
            pl.BlockSpec((28, 28, _BT), lambda i: (0, 0, i)),
            pl.BlockSpec((192, 140), lambda i: (0, 0)),
            pl.BlockSpec((256, 480), lambda i: (0, 0)),
            pl.BlockSpec((512, 896), lambda i: (0, 0)),
            pl.BlockSpec((10, 512), lambda i: (0, 0)),
            pl.BlockSpec((768, _BT), lambda i: (0, 0)),
        ],
        out_specs=pl.BlockSpec((10, _BT), lambda i: (0, i)),
        scratch_shapes=[
            pltpu.VMEM((18, 96, _BT), _F32),
        ],
        compiler_params=pltpu.CompilerParams(
            dimension_semantics=("parallel",),
            vmem_limit_bytes=64 * 1024 * 1024,
        ),
    )(xT, w1c, w2c, w1m, w2m, bias)
    return jnp.transpose(out)


# speedup vs baseline: 1.0755x; 1.0218x over previous
import jax
import jax.numpy as jnp
from jax.experimental import pallas as pl
from jax.experimental.pallas import tpu as pltpu

_F32 = jnp.float32
_BT = 256


def _lenet_kernel(x_ref, w1c_ref, w2c_ref, w1m_ref, w2m_ref, bias_ref,
                  o_ref, qp):
    w1c = w1c_ref[...]
    bias = bias_ref[...]
    b1p = bias[0:96]
    b2p = bias[96:224]
    b1m = bias[224:736]
    b2m = bias[736:746]
    z28 = jnp.zeros((28, _BT), jnp.bfloat16)
    slab = [z28, z28] + [x_ref[j] for j in range(28)] + [z28, z28]
    qp[0:2] = jnp.zeros((2, 96, _BT), _F32)
    qp[16:18] = jnp.zeros((2, 96, _BT), _F32)
    for w2 in range(14):
        ra = jnp.concatenate([slab[2 * w2 + kw] for kw in range(5)], axis=0)
        rb = jnp.concatenate([slab[2 * w2 + 1 + kw] for kw in range(5)],
                             axis=0)
        ya = jnp.dot(w1c, ra, preferred_element_type=_F32)
        yb = jnp.dot(w1c, rb, preferred_element_type=_F32)
        m = jnp.maximum(jnp.maximum(ya[:96], ya[96:]),
                        jnp.maximum(yb[:96], yb[96:]))
        qp[2 + w2] = jnp.maximum(m + b1p, 0.0)

    w2c = w2c_ref[...]
    cols = []
    for w4 in range(7):
        ra = jnp.concatenate([qp[2 * w4 + kw] for kw in range(5)], axis=0)
        rb = jnp.concatenate([qp[2 * w4 + 1 + kw] for kw in range(5)], axis=0)
        ya = jnp.dot(w2c, ra, preferred_element_type=_F32)
        yb = jnp.dot(w2c, rb, preferred_element_type=_F32)
        m = jnp.maximum(jnp.maximum(ya[:128], ya[128:]),
                        jnp.maximum(yb[:128], yb[128:]))
        cols.append(jnp.maximum(m + b2p, 0.0))
    f = jnp.concatenate(cols, axis=0)

    h = jnp.dot(w1m_ref[...], f, preferred_element_type=_F32)
    h = jnp.maximum(h + b1m, 0.0)
    y = jnp.dot(w2m_ref[...], h, preferred_element_type=_F32) + b2m
    m = jnp.max(y, axis=0, keepdims=True)
    z = y - m
    o_ref[...] = z - jnp.log(jnp.sum(jnp.exp(z), axis=0, keepdims=True))


def _band_part(wk, p, n_out, real_out, n_in, real_in):
    oc_n, ic_n = wk.shape[0], wk.shape[1]
    h2 = jnp.arange(n_out)
    hp = jnp.arange(n_in)
    band = jnp.zeros((oc_n, n_out, 5, ic_n, n_in), _F32)
    for kh in range(5):
        e = ((hp[None, :] - (2 * h2[:, None] + p) == kh - 2)
             & (h2[:, None] < real_out)
             & (hp[None, :] < real_in)).astype(_F32)
        band = band + (jnp.transpose(wk[:, :, kh, :], (0, 2, 1))
                       [:, None, :, :, None] * e[None, :, None, None, :])
    return band.reshape(oc_n * n_out, 5 * ic_n * n_in)


def _band_eo(wk, n_out, real_out, n_in, real_in):
    return jnp.concatenate(
        [_band_part(wk, 0, n_out, real_out, n_in, real_in),
         _band_part(wk, 1, n_out, real_out, n_in, real_in)], axis=0)


def kernel(x_nchw, conv1_wmat, conv1_b, conv2_wmat, conv2_b,
           fc1_wt, fc1_b, fc2_wt, fc2_b):
    B = x_nchw.shape[0]
    grid = B // _BT

    xT = jnp.transpose(x_nchw.reshape(B, 28, 28).astype(jnp.bfloat16),
                       (2, 1, 0))

    w1 = jnp.transpose(conv1_wmat[:25, :6]).reshape(6, 1, 5, 5)
    w2 = jnp.transpose(conv2_wmat[:150, :16].reshape(5, 5, 6, 16),
                       (3, 2, 0, 1))
    w1c = _band_eo(w1, 16, 14, 28, 28).astype(jnp.bfloat16)
    w2c = _band_eo(w2, 8, 7, 16, 14)
    hm1 = (jnp.arange(16) < 14).astype(_F32)
    hm2 = (jnp.arange(8) < 7).astype(_F32)
    bias = jnp.concatenate([
        (conv1_b[:6, None] * hm1).reshape(96),
        (conv2_b[:16, None] * hm2).reshape(128),
        fc1_b, fc2_b, jnp.zeros((22,), _F32)])
    bias = jnp.broadcast_to(bias[:, None], (768, _BT))

    w1m = fc1_wt.reshape(7, 7, 16, 512).transpose(1, 2, 0, 3)
    w1m = jnp.pad(w1m, ((0, 0), (0, 0), (0, 1), (0, 0)))
    w1m = w1m.reshape(896, 512).T
    w2m = fc2_wt.T

    out = pl.pallas_call(
        _lenet_kernel,
        out_shape=jax.ShapeDtypeStruct((10, B), _F32),
        grid=(grid,),
        in_specs=[
            pl.BlockSpec((28, 28, _BT), lambda i: (0, 0, i)),
            pl.BlockSpec((192, 140), lambda i: (0, 0)),
            pl.BlockSpec((256, 480), lambda i: (0, 0)),
            pl.BlockSpec((512, 896), lambda i: (0, 0)),
            pl.BlockSpec((10, 512), lambda i: (0, 0)),
            pl.BlockSpec((768, _BT), lambda i: (0, 0)),
        ],
        out_specs=pl.BlockSpec((10, _BT), lambda i: (0, i)),
        scratch_shapes=[
            pltpu.VMEM((18, 96, _BT), _F32),
        ],
        compiler_params=pltpu.CompilerParams(
            dimension_semantics=("parallel",),
            vmem_limit_bytes=64 * 1024 * 1024,
        ),
    )(xT, w1c, w2c, w1m, w2m, bias)
    return jnp.transpose(out)
